# baseline (device time: 92097 ns/iter reference)
import jax
import jax.numpy as jnp
from jax import lax
from jax.experimental import pallas as pl
from jax.experimental.pallas import tpu as pltpu

N_DEV = 4
SQ, HQ, DH = 2048, 8, 128
DM = HQ * DH
PW = DM + 128
A = 8
QSEL = A * 64
SCALE = 0.08838834764831843
F32 = jnp.float32
BF16 = jnp.bfloat16

_MESH = pl.DeviceIdType.MESH


def _body(x_ref, wq_ref, kt_ref, vg_ref, wo_ref, out_ref,
          pay_ref, pay_rx, comm,
          pc_ssem, pc_rsem, ag_ssem, ag_rsem):
    me = lax.axis_index("i")

    bsem = pltpu.get_barrier_semaphore()
    for off in (1, 2, 3):
        pl.semaphore_signal(
            bsem, inc=1, device_id=((me + off) % N_DEV,),
            device_id_type=_MESH,
        )
    pl.semaphore_wait(bsem, 3)

    sends = []

    for slot, off in enumerate((2, 1, 3, 0)):
        r = (me + off) % N_DEV
        xr = x_ref[pl.ds(r, 1)].reshape(QSEL, DM)
        ktr = kt_ref[pl.ds(r, 1)].reshape(HQ, DH, QSEL)
        vgr = vg_ref[pl.ds(r, 1)].reshape(HQ, QSEL, DH)
        ctx_parts = []
        l_cols = []
        for h in range(HQ):
            qh = lax.dot_general(
                xr, wq_ref[h], (((1,), (0,)), ((), ())),
                preferred_element_type=F32,
            )
            qh = (qh * SCALE).astype(BF16)
            s = lax.dot_general(
                qh, ktr[h], (((1,), (0,)), ((), ())),
                preferred_element_type=F32,
            )
            e = jnp.exp(s)
            l_cols.append(jnp.sum(e, axis=1, keepdims=True).astype(BF16))
            c = lax.dot_general(
                e.astype(BF16), vgr[h], (((1,), (0,)), ((), ())),
                preferred_element_type=F32,
            )
            ctx_parts.append(c.astype(BF16))
        pay_val = jnp.concatenate(
            ctx_parts + l_cols + [jnp.zeros((QSEL, 128 - HQ), BF16)], axis=1
        )

        if off == 0:
            pay_rx[pl.ds(me, 1)] = pay_val[None]
        else:
            pay_ref[pl.ds(r, 1)] = pay_val[None]
            rdma = pltpu.make_async_remote_copy(
                src_ref=pay_ref.at[r],
                dst_ref=pay_rx.at[me],
                send_sem=pc_ssem.at[slot],
                recv_sem=pc_rsem.at[me],
                device_id=(r,), device_id_type=_MESH,
            )
            rdma.start()
            sends.append(rdma)

    for off in (1, 3, 2):
        p = (me + off) % N_DEV
        pltpu.make_async_remote_copy(
            src_ref=pay_ref.at[p], dst_ref=pay_rx.at[p],
            send_sem=pc_ssem.at[0], recv_sem=pc_rsem.at[p],
            device_id=(p,), device_id_type=_MESH,
        ).wait_recv()

    ctx_parts = []
    for h in range(HQ):
        num = pay_rx[0, :, h * DH:(h + 1) * DH].astype(F32)
        den = pay_rx[0, :, DM + h:DM + h + 1].astype(F32)
        for p in range(1, N_DEV):
            num = num + pay_rx[p, :, h * DH:(h + 1) * DH].astype(F32)
            den = den + pay_rx[p, :, DM + h:DM + h + 1].astype(F32)
        ctx_parts.append((num / den).astype(BF16))
    ctx = jnp.concatenate(ctx_parts, axis=1)

    comm[pl.ds(me, 1)] = ctx[None]
    for slot, off in enumerate((2, 1, 3)):
        dst = (me + off) % N_DEV
        rdma = pltpu.make_async_remote_copy(
            src_ref=comm.at[me],
            dst_ref=comm.at[me],
            send_sem=ag_ssem.at[slot],
            recv_sem=ag_rsem.at[me],
            device_id=(dst,), device_id_type=_MESH,
        )
        rdma.start()
        sends.append(rdma)

    wo = wo_ref[...]

    def chunk_out(p, chunk):
        mm = lax.dot_general(
            chunk, wo, (((1,), (0,)), ((), ())),
            preferred_element_type=F32,
        )
        for a in range(A):
            out_ref[pl.ds(64 * p + 256 * a, 64), :] = mm[a * 64:(a + 1) * 64, :]

    chunk_out(me, ctx)
    for off in (1, 3, 2):
        p = (me + off) % N_DEV
        pltpu.make_async_remote_copy(
            src_ref=comm.at[p], dst_ref=comm.at[p],
            send_sem=ag_ssem.at[0], recv_sem=ag_rsem.at[p],
            device_id=(p,), device_id_type=_MESH,
        ).wait_recv()
        chunk_out(p, comm[pl.ds(p, 1)].reshape(QSEL, DM))

    for rdma in sends:
        rdma.wait_send()


def kernel(x, Wq, K_ext, V_ext, Wo):
    xg = (
        x[0].astype(BF16)
        .reshape(A, N_DEV, 64, DM)
        .transpose(1, 0, 2, 3)
        .reshape(N_DEV, QSEL, DM)
    )

    wqr = Wq.astype(BF16).reshape(DM, HQ, DH).transpose(1, 0, 2)

    k5 = K_ext[0].astype(BF16).reshape(A, N_DEV, 64, HQ, DH)
    ktg = k5.transpose(1, 3, 4, 0, 2).reshape(N_DEV, HQ, DH, QSEL)
    v5 = V_ext[0].astype(BF16).reshape(A, N_DEV, 64, HQ, DH)
    vgg = v5.transpose(1, 3, 0, 2, 4).reshape(N_DEV, HQ, QSEL, DH)

    wo = Wo.astype(BF16)

    out = pl.pallas_call(
        _body,
        out_shape=jax.ShapeDtypeStruct((SQ, DM), F32),
        in_specs=[pl.BlockSpec(memory_space=pltpu.VMEM)] * 5,
        out_specs=pl.BlockSpec(memory_space=pltpu.VMEM),
        scratch_shapes=[
            pltpu.VMEM((N_DEV, QSEL, PW), BF16),
            pltpu.VMEM((N_DEV, QSEL, PW), BF16),
            pltpu.VMEM((N_DEV, QSEL, DM), BF16),
            pltpu.SemaphoreType.DMA((3,)),
            pltpu.SemaphoreType.DMA((N_DEV,)),
            pltpu.SemaphoreType.DMA((3,)),
            pltpu.SemaphoreType.DMA((N_DEV,)),
        ],
        compiler_params=pltpu.CompilerParams(collective_id=0),
    )(xg, wqr, ktg, vgg, wo)
    return out.reshape(1, SQ, DM)


# device time: 89869 ns/iter; 1.0248x vs baseline; 1.0248x over previous
import jax
import jax.numpy as jnp
from jax import lax
from jax.experimental import pallas as pl
from jax.experimental.pallas import tpu as pltpu

N_DEV = 4
SQ, HQ, DH = 2048, 8, 128
DM = HQ * DH
PW = DM + 128
A = 8
QSEL = A * 64
QH = QSEL // 2
SCALE = 0.08838834764831843
F32 = jnp.float32
BF16 = jnp.bfloat16

_MESH = pl.DeviceIdType.MESH


def _body(x_ref, wq_ref, kt_ref, vg_ref, wo_ref, out_ref,
          pay_ref, pay_rx, comm,
          pc_ssem, pc_rsem, ag_ssem, ag_rsem):
    me = lax.axis_index("i")

    bsem = pltpu.get_barrier_semaphore()
    for off in (1, 2, 3):
        pl.semaphore_signal(
            bsem, inc=1, device_id=((me + off) % N_DEV,),
            device_id_type=_MESH,
        )
    pl.semaphore_wait(bsem, 3)

    sends = []

    def partial(half, slot, off):
        r = (me + off) % N_DEV
        xr = x_ref[pl.ds(r, 1), pl.ds(half * QH, QH)].reshape(QH, DM)
        ktr = kt_ref[pl.ds(r, 1)].reshape(HQ, DH, QSEL)
        vgr = vg_ref[pl.ds(r, 1)].reshape(HQ, QSEL, DH)
        ctx_parts = []
        l_cols = []
        for h in range(HQ):
            qh = lax.dot_general(
                xr, wq_ref[h], (((1,), (0,)), ((), ())),
                preferred_element_type=F32,
            )
            qh = (qh * SCALE).astype(BF16)
            s = lax.dot_general(
                qh, ktr[h], (((1,), (0,)), ((), ())),
                preferred_element_type=F32,
            )
            e = jnp.exp(s)
            l_cols.append(jnp.sum(e, axis=1, keepdims=True).astype(BF16))
            c = lax.dot_general(
                e.astype(BF16), vgr[h], (((1,), (0,)), ((), ())),
                preferred_element_type=F32,
            )
            ctx_parts.append(c.astype(BF16))
        pay_val = jnp.concatenate(
            ctx_parts + l_cols + [jnp.zeros((QH, 128 - HQ), BF16)], axis=1
        )

        if off == 0:
            pay_rx[pl.ds(4 * half + me, 1)] = pay_val[None]
        else:
            pay_ref[pl.ds(4 * half + r, 1)] = pay_val[None]
            rdma = pltpu.make_async_remote_copy(
                src_ref=pay_ref.at[4 * half + r],
                dst_ref=pay_rx.at[4 * half + me],
                send_sem=pc_ssem.at[3 * half + slot],
                recv_sem=pc_rsem.at[4 * half + me],
                device_id=(r,), device_id_type=_MESH,
            )
            rdma.start()
            sends.append(rdma)

    def merge_and_exchange(half):
        for off in (1, 3, 2):
            p = (me + off) % N_DEV
            pltpu.make_async_remote_copy(
                src_ref=pay_ref.at[4 * half + p],
                dst_ref=pay_rx.at[4 * half + p],
                send_sem=pc_ssem.at[0], recv_sem=pc_rsem.at[4 * half + p],
                device_id=(p,), device_id_type=_MESH,
            ).wait_recv()
        ctx_parts = []
        for h in range(HQ):
            num = pay_rx[4 * half, :, h * DH:(h + 1) * DH].astype(F32)
            den = pay_rx[4 * half, :, DM + h:DM + h + 1].astype(F32)
            for p in range(1, N_DEV):
                num = num + pay_rx[4 * half + p, :, h * DH:(h + 1) * DH].astype(F32)
                den = den + pay_rx[4 * half + p, :, DM + h:DM + h + 1].astype(F32)
            ctx_parts.append((num / den).astype(BF16))
        ctx = jnp.concatenate(ctx_parts, axis=1)

        comm[pl.ds(4 * half + me, 1)] = ctx[None]
        for slot, off in enumerate((2, 1, 3)):
            dst = (me + off) % N_DEV
            rdma = pltpu.make_async_remote_copy(
                src_ref=comm.at[4 * half + me],
                dst_ref=comm.at[4 * half + me],
                send_sem=ag_ssem.at[3 * half + slot],
                recv_sem=ag_rsem.at[4 * half + me],
                device_id=(dst,), device_id_type=_MESH,
            )
            rdma.start()
            sends.append(rdma)
        chunk_out(me, half, ctx)

    def chunk_out(p, half, chunk):
        mm = lax.dot_general(
            chunk, wo_ref[...], (((1,), (0,)), ((), ())),
            preferred_element_type=F32,
        )
        for aa in range(4):
            a = 4 * half + aa
            out_ref[pl.ds(64 * p + 256 * a, 64), :] = mm[aa * 64:(aa + 1) * 64, :]

    for half, slot, off in ((0, 0, 2), (0, 1, 1), (0, 2, 3), (0, -1, 0),
                            (1, 0, 2), (1, 1, 1), (1, 2, 3)):
        partial(half, slot, off)
    merge_and_exchange(0)
    partial(1, -1, 0)
    merge_and_exchange(1)

    for half in (0, 1):
        for off in (1, 3, 2):
            p = (me + off) % N_DEV
            pltpu.make_async_remote_copy(
                src_ref=comm.at[4 * half + p], dst_ref=comm.at[4 * half + p],
                send_sem=ag_ssem.at[0], recv_sem=ag_rsem.at[4 * half + p],
                device_id=(p,), device_id_type=_MESH,
            ).wait_recv()
            chunk_out(p, half, comm[pl.ds(4 * half + p, 1)].reshape(QH, DM))

    for rdma in sends:
        rdma.wait_send()


def kernel(x, Wq, K_ext, V_ext, Wo):
    xg = (
        x[0].astype(BF16)
        .reshape(A, N_DEV, 64, DM)
        .transpose(1, 0, 2, 3)
        .reshape(N_DEV, QSEL, DM)
    )

    wqr = Wq.astype(BF16).reshape(DM, HQ, DH).transpose(1, 0, 2)

    k5 = K_ext[0].astype(BF16).reshape(A, N_DEV, 64, HQ, DH)
    ktg = k5.transpose(1, 3, 4, 0, 2).reshape(N_DEV, HQ, DH, QSEL)
    v5 = V_ext[0].astype(BF16).reshape(A, N_DEV, 64, HQ, DH)
    vgg = v5.transpose(1, 3, 0, 2, 4).reshape(N_DEV, HQ, QSEL, DH)

    wo = Wo.astype(BF16)

    out = pl.pallas_call(
        _body,
        out_shape=jax.ShapeDtypeStruct((SQ, DM), F32),
        in_specs=[pl.BlockSpec(memory_space=pltpu.VMEM)] * 5,
        out_specs=pl.BlockSpec(memory_space=pltpu.VMEM),
        scratch_shapes=[
            pltpu.VMEM((2 * N_DEV, QH, PW), BF16),
            pltpu.VMEM((2 * N_DEV, QH, PW), BF16),
            pltpu.VMEM((2 * N_DEV, QH, DM), BF16),
            pltpu.SemaphoreType.DMA((6,)),
            pltpu.SemaphoreType.DMA((2 * N_DEV,)),
            pltpu.SemaphoreType.DMA((6,)),
            pltpu.SemaphoreType.DMA((2 * N_DEV,)),
        ],
        compiler_params=pltpu.CompilerParams(collective_id=0),
    )(xg, wqr, ktg, vgg, wo)
    return out.reshape(1, SQ, DM)


# device time: 83933 ns/iter; 1.0973x vs baseline; 1.0707x over previous
import jax
import jax.numpy as jnp
from jax import lax
from jax.experimental import pallas as pl
from jax.experimental.pallas import tpu as pltpu

N_DEV = 4
SQ, HQ, DH = 2048, 8, 128
DM = HQ * DH
PW = DM + 128
A = 8
QSEL = A * 64
QH = QSEL // 2
SCALE = 0.08838834764831843
F32 = jnp.float32
BF16 = jnp.bfloat16

_MESH = pl.DeviceIdType.MESH


def _body(x_ref, wq_ref, kt_ref, vg_ref, wo_ref, out_ref,
          pay_ref, pay_rx, comm,
          pc_ssem, pc_rsem, ag_ssem, ag_rsem):
    me = lax.axis_index("i")

    bsem = pltpu.get_barrier_semaphore()
    for off in (1, 2, 3):
        pl.semaphore_signal(
            bsem, inc=1, device_id=((me + off) % N_DEV,),
            device_id_type=_MESH,
        )
    pl.semaphore_wait(bsem, 3)

    sends = []

    def partial(half, slot, off):
        r = (me + off) % N_DEV
        xr = x_ref[pl.ds(r, 1), pl.ds(half * QH, QH)].reshape(QH, DM)
        ktr = kt_ref[pl.ds(r, 1)].reshape(HQ, DH, QSEL)
        vgr = vg_ref[pl.ds(r, 1)].reshape(HQ, QSEL, DH)
        qall = lax.dot_general(
            xr, wq_ref[...], (((1,), (0,)), ((), ())),
            preferred_element_type=F32,
        )
        qall = (qall * SCALE).astype(BF16)
        ctx_parts = []
        l_cols = []
        for h in range(HQ):
            s = lax.dot_general(
                qall[:, h * DH:(h + 1) * DH], ktr[h], (((1,), (0,)), ((), ())),
                preferred_element_type=F32,
            )
            e = jnp.exp(s)
            l_cols.append(jnp.sum(e, axis=1, keepdims=True).astype(BF16))
            c = lax.dot_general(
                e.astype(BF16), vgr[h], (((1,), (0,)), ((), ())),
                preferred_element_type=F32,
            )
            ctx_parts.append(c.astype(BF16))
        pay_val = jnp.concatenate(
            ctx_parts + l_cols + [jnp.zeros((QH, 128 - HQ), BF16)], axis=1
        )

        if off == 0:
            pay_rx[pl.ds(4 * half + me, 1)] = pay_val[None]
        else:
            pay_ref[pl.ds(4 * half + r, 1)] = pay_val[None]
            rdma = pltpu.make_async_remote_copy(
                src_ref=pay_ref.at[4 * half + r],
                dst_ref=pay_rx.at[4 * half + me],
                send_sem=pc_ssem.at[3 * half + slot],
                recv_sem=pc_rsem.at[4 * half + me],
                device_id=(r,), device_id_type=_MESH,
            )
            rdma.start()
            sends.append(rdma)

    def merge_and_exchange(half):
        for off in (1, 3, 2):
            p = (me + off) % N_DEV
            pltpu.make_async_remote_copy(
                src_ref=pay_ref.at[4 * half + p],
                dst_ref=pay_rx.at[4 * half + p],
                send_sem=pc_ssem.at[0], recv_sem=pc_rsem.at[4 * half + p],
                device_id=(p,), device_id_type=_MESH,
            ).wait_recv()
        ctx_parts = []
        for h in range(HQ):
            num = pay_rx[4 * half, :, h * DH:(h + 1) * DH].astype(F32)
            den = pay_rx[4 * half, :, DM + h:DM + h + 1].astype(F32)
            for p in range(1, N_DEV):
                num = num + pay_rx[4 * half + p, :, h * DH:(h + 1) * DH].astype(F32)
                den = den + pay_rx[4 * half + p, :, DM + h:DM + h + 1].astype(F32)
            ctx_parts.append((num / den).astype(BF16))
        ctx = jnp.concatenate(ctx_parts, axis=1)

        comm[pl.ds(4 * half + me, 1)] = ctx[None]
        for slot, off in enumerate((2, 1, 3)):
            dst = (me + off) % N_DEV
            rdma = pltpu.make_async_remote_copy(
                src_ref=comm.at[4 * half + me],
                dst_ref=comm.at[4 * half + me],
                send_sem=ag_ssem.at[3 * half + slot],
                recv_sem=ag_rsem.at[4 * half + me],
                device_id=(dst,), device_id_type=_MESH,
            )
            rdma.start()
            sends.append(rdma)
        chunk_out(me, half, ctx)

    def chunk_out(p, half, chunk):
        mm = lax.dot_general(
            chunk, wo_ref[...], (((1,), (0,)), ((), ())),
            preferred_element_type=F32,
        )
        for aa in range(4):
            a = 4 * half + aa
            out_ref[pl.ds(64 * p + 256 * a, 64), :] = mm[aa * 64:(aa + 1) * 64, :]

    for half, slot, off in ((0, 0, 2), (0, 1, 1), (0, 2, 3), (0, -1, 0),
                            (1, 0, 2), (1, 1, 1), (1, 2, 3)):
        partial(half, slot, off)
    merge_and_exchange(0)
    partial(1, -1, 0)
    merge_and_exchange(1)

    for half in (0, 1):
        for off in (1, 3, 2):
            p = (me + off) % N_DEV
            pltpu.make_async_remote_copy(
                src_ref=comm.at[4 * half + p], dst_ref=comm.at[4 * half + p],
                send_sem=ag_ssem.at[0], recv_sem=ag_rsem.at[4 * half + p],
                device_id=(p,), device_id_type=_MESH,
            ).wait_recv()
            chunk_out(p, half, comm[pl.ds(4 * half + p, 1)].reshape(QH, DM))

    for rdma in sends:
        rdma.wait_send()


def kernel(x, Wq, K_ext, V_ext, Wo):
    xg = (
        x[0].astype(BF16)
        .reshape(A, N_DEV, 64, DM)
        .transpose(1, 0, 2, 3)
        .reshape(N_DEV, QSEL, DM)
    )

    wqr = Wq.astype(BF16)

    k5 = K_ext[0].astype(BF16).reshape(A, N_DEV, 64, HQ, DH)
    ktg = k5.transpose(1, 3, 4, 0, 2).reshape(N_DEV, HQ, DH, QSEL)
    v5 = V_ext[0].astype(BF16).reshape(A, N_DEV, 64, HQ, DH)
    vgg = v5.transpose(1, 3, 0, 2, 4).reshape(N_DEV, HQ, QSEL, DH)

    wo = Wo.astype(BF16)

    out = pl.pallas_call(
        _body,
        out_shape=jax.ShapeDtypeStruct((SQ, DM), F32),
        in_specs=[pl.BlockSpec(memory_space=pltpu.VMEM)] * 5,
        out_specs=pl.BlockSpec(memory_space=pltpu.VMEM),
        scratch_shapes=[
            pltpu.VMEM((2 * N_DEV, QH, PW), BF16),
            pltpu.VMEM((2 * N_DEV, QH, PW), BF16),
            pltpu.VMEM((2 * N_DEV, QH, DM), BF16),
            pltpu.SemaphoreType.DMA((6,)),
            pltpu.SemaphoreType.DMA((2 * N_DEV,)),
            pltpu.SemaphoreType.DMA((6,)),
            pltpu.SemaphoreType.DMA((2 * N_DEV,)),
        ],
        compiler_params=pltpu.CompilerParams(collective_id=0),
    )(xg, wqr, ktg, vgg, wo)
    return out.reshape(1, SQ, DM)


# device time: 81565 ns/iter; 1.1291x vs baseline; 1.0290x over previous
import jax
import jax.numpy as jnp
from jax import lax
from jax.experimental import pallas as pl
from jax.experimental.pallas import tpu as pltpu

N_DEV = 4
SQ, HQ, DH = 2048, 8, 128
DM = HQ * DH
PW = DM + HQ
A = 8
QSEL = A * 64
QH = QSEL // 2
SCALE = 0.08838834764831843
F32 = jnp.float32
BF16 = jnp.bfloat16

_MESH = pl.DeviceIdType.MESH


def _body(x_ref, wq_ref, kt_ref, vg_ref, wo_ref, out_ref,
          pay_ref, pay_rx, comm,
          pc_ssem, pc_rsem, ag_ssem, ag_rsem):
    me = lax.axis_index("i")

    bsem = pltpu.get_barrier_semaphore()
    for off in (1, 2, 3):
        pl.semaphore_signal(
            bsem, inc=1, device_id=((me + off) % N_DEV,),
            device_id_type=_MESH,
        )
    pl.semaphore_wait(bsem, 3)

    sends = []

    def partial(half, slot, off):
        r = (me + off) % N_DEV
        xr = x_ref[pl.ds(r, 1), pl.ds(half * QH, QH)].reshape(QH, DM)
        ktr = kt_ref[pl.ds(r, 1)].reshape(HQ, DH, QSEL)
        vgr = vg_ref[pl.ds(r, 1)].reshape(HQ, QSEL, DH)
        qall = lax.dot_general(
            xr, wq_ref[...], (((1,), (0,)), ((), ())),
            preferred_element_type=F32,
        )
        qall = (qall * SCALE).astype(BF16)
        ctx_parts = []
        l_cols = []
        for h in range(HQ):
            s = lax.dot_general(
                qall[:, h * DH:(h + 1) * DH], ktr[h], (((1,), (0,)), ((), ())),
                preferred_element_type=F32,
            )
            e = jnp.exp(s)
            l_cols.append(jnp.sum(e, axis=1, keepdims=True).astype(BF16))
            c = lax.dot_general(
                e.astype(BF16), vgr[h], (((1,), (0,)), ((), ())),
                preferred_element_type=F32,
            )
            ctx_parts.append(c.astype(BF16))
        pay_val = jnp.concatenate(ctx_parts + l_cols, axis=1)

        if off == 0:
            pay_rx[pl.ds(4 * half + me, 1)] = pay_val[None]
        else:
            pay_ref[pl.ds(4 * half + r, 1)] = pay_val[None]
            rdma = pltpu.make_async_remote_copy(
                src_ref=pay_ref.at[4 * half + r],
                dst_ref=pay_rx.at[4 * half + me],
                send_sem=pc_ssem.at[3 * half + slot],
                recv_sem=pc_rsem.at[4 * half + me],
                device_id=(r,), device_id_type=_MESH,
            )
            rdma.start()
            sends.append(rdma)

    def merge_and_exchange(half):
        for off in (1, 3, 2):
            p = (me + off) % N_DEV
            pltpu.make_async_remote_copy(
                src_ref=pay_ref.at[4 * half + p],
                dst_ref=pay_rx.at[4 * half + p],
                send_sem=pc_ssem.at[0], recv_sem=pc_rsem.at[4 * half + p],
                device_id=(p,), device_id_type=_MESH,
            ).wait_recv()
        ctx_parts = []
        for h in range(HQ):
            num = pay_rx[4 * half, :, h * DH:(h + 1) * DH].astype(F32)
            den = pay_rx[4 * half, :, DM + h:DM + h + 1].astype(F32)
            for p in range(1, N_DEV):
                num = num + pay_rx[4 * half + p, :, h * DH:(h + 1) * DH].astype(F32)
                den = den + pay_rx[4 * half + p, :, DM + h:DM + h + 1].astype(F32)
            ctx_parts.append((num / den).astype(BF16))
        ctx = jnp.concatenate(ctx_parts, axis=1)

        mm = lax.dot_general(
            ctx, wo_ref[...], (((1,), (0,)), ((), ())),
            preferred_element_type=F32,
        )
        comm[pl.ds(4 * half + me, 1)] = mm.astype(BF16)[None]
        for slot, off in enumerate((2, 1, 3)):
            dst = (me + off) % N_DEV
            rdma = pltpu.make_async_remote_copy(
                src_ref=comm.at[4 * half + me],
                dst_ref=comm.at[4 * half + me],
                send_sem=ag_ssem.at[3 * half + slot],
                recv_sem=ag_rsem.at[4 * half + me],
                device_id=(dst,), device_id_type=_MESH,
            )
            rdma.start()
            sends.append(rdma)
        scatter_out(me, half, mm)

    def scatter_out(p, half, mm):
        for aa in range(4):
            a = 4 * half + aa
            out_ref[pl.ds(64 * p + 256 * a, 64), :] = (
                mm[aa * 64:(aa + 1) * 64, :].astype(F32)
            )

    for half, slot, off in ((0, 0, 2), (0, 1, 1), (0, 2, 3), (0, -1, 0),
                            (1, 0, 2), (1, 1, 1), (1, 2, 3)):
        partial(half, slot, off)
    merge_and_exchange(0)
    partial(1, -1, 0)
    merge_and_exchange(1)

    for half in (0, 1):
        for off in (1, 3, 2):
            p = (me + off) % N_DEV
            pltpu.make_async_remote_copy(
                src_ref=comm.at[4 * half + p], dst_ref=comm.at[4 * half + p],
                send_sem=ag_ssem.at[0], recv_sem=ag_rsem.at[4 * half + p],
                device_id=(p,), device_id_type=_MESH,
            ).wait_recv()
            scatter_out(p, half, comm[pl.ds(4 * half + p, 1)].reshape(QH, DM))

    for rdma in sends:
        rdma.wait_send()


def kernel(x, Wq, K_ext, V_ext, Wo):
    xg = (
        x[0].astype(BF16)
        .reshape(A, N_DEV, 64, DM)
        .transpose(1, 0, 2, 3)
        .reshape(N_DEV, QSEL, DM)
    )

    wqr = Wq.astype(BF16)

    k5 = K_ext[0].astype(BF16).reshape(A, N_DEV, 64, HQ, DH)
    ktg = k5.transpose(1, 3, 4, 0, 2).reshape(N_DEV, HQ, DH, QSEL)
    v5 = V_ext[0].astype(BF16).reshape(A, N_DEV, 64, HQ, DH)
    vgg = v5.transpose(1, 3, 0, 2, 4).reshape(N_DEV, HQ, QSEL, DH)

    wo = Wo.astype(BF16)

    out = pl.pallas_call(
        _body,
        out_shape=jax.ShapeDtypeStruct((SQ, DM), F32),
        in_specs=[pl.BlockSpec(memory_space=pltpu.VMEM)] * 5,
        out_specs=pl.BlockSpec(memory_space=pltpu.VMEM),
        scratch_shapes=[
            pltpu.VMEM((2 * N_DEV, QH, PW), BF16),
            pltpu.VMEM((2 * N_DEV, QH, PW), BF16),
            pltpu.VMEM((2 * N_DEV, QH, DM), BF16),
            pltpu.SemaphoreType.DMA((6,)),
            pltpu.SemaphoreType.DMA((2 * N_DEV,)),
            pltpu.SemaphoreType.DMA((6,)),
            pltpu.SemaphoreType.DMA((2 * N_DEV,)),
        ],
        compiler_params=pltpu.CompilerParams(collective_id=0),
    )(xg, wqr, ktg, vgg, wo)
    return out.reshape(1, SQ, DM)
